# baseline (speedup 1.0000x reference)
import functools

import jax
import jax.numpy as jnp
from jax import lax
from jax.experimental import pallas as pl
from jax.experimental.pallas import tpu as pltpu
from jax.experimental.pallas import tpu_sc as plsc

BATCH = 16384
NUM_FIELDS = 26
EMBED_DIM = 16
NUM_EMB = 2600000
NUM_GRP = NUM_EMB // 8
TOTAL = BATCH * NUM_FIELDS

_info = plsc.get_sparse_core_info()
_NC, _NS = _info.num_cores, _info.num_subcores
_NW = _NC * _NS
_BATCH_PER_W = BATCH // _NW
_CB = 8
_CHUNK = _CB * NUM_FIELDS
_NSTEP = _BATCH_PER_W // _CB

_mesh = plsc.VectorSubcoreMesh(core_axis_name="c", subcore_axis_name="s")


@functools.partial(
    pl.kernel,
    mesh=_mesh,
    out_type=jax.ShapeDtypeStruct((BATCH, 32, 128), jnp.float32),
    scratch_types=[
        pltpu.VMEM((_CHUNK,), jnp.int32),
        pltpu.VMEM((_CHUNK,), jnp.int32),
        pltpu.VMEM((_CHUNK, 128), jnp.float32),
        pltpu.VMEM((_CHUNK, EMBED_DIM), jnp.float32),
        pltpu.SemaphoreType.DMA,
    ],
    compiler_params=pltpu.CompilerParams(
        use_tc_tiling_on_sc=False, needs_layout_passes=False),
)
def _gather_rows(table_hbm, idx_hbm, out_hbm, x_v, g_v, grp_v, rows_v, sem):
    wid = lax.axis_index("s") * _NC + lax.axis_index("c")
    lanes = lax.iota(jnp.int32, 16)

    def step(i, carry):
        b0 = wid * _BATCH_PER_W + i * _CB
        r0 = b0 * NUM_FIELDS
        pltpu.sync_copy(idx_hbm.at[pl.ds(r0, _CHUNK)], x_v)
        for j in range(_CHUNK // 16):
            v = x_v[pl.ds(j * 16, 16)]
            g_v[pl.ds(j * 16, 16)] = lax.shift_right_logical(v, 3)
        pltpu.async_copy(table_hbm.at[g_v], grp_v, sem).wait()
        for j in range(_CHUNK // 16):
            i_vec = j * 16 + lanes
            v = x_v[pl.ds(j * 16, 16)]
            c0 = jnp.bitwise_and(v, 7) * 16
            for d in range(EMBED_DIM):
                val = plsc.load_gather(grp_v, [i_vec, c0 + d])
                plsc.store_scatter(rows_v, [i_vec, jnp.full((16,), d,
                                                            jnp.int32)], val)
        copies = [
            pltpu.async_copy(
                rows_v.at[pl.ds(cb * NUM_FIELDS, NUM_FIELDS), :],
                out_hbm.at[b0 + cb, pl.ds(0, NUM_FIELDS),
                           pl.ds(0, EMBED_DIM)],
                sem)
            for cb in range(_CB)
        ]
        for cp in copies:
            cp.wait()
        return carry

    lax.fori_loop(0, _NSTEP, step, 0)


def kernel(table, x):
    t128 = table.reshape(NUM_GRP, 128)
    flat = x.reshape(TOTAL)
    out_pad = _gather_rows(t128, flat)
    return out_pad[:, :NUM_FIELDS, :EMBED_DIM]


# speedup vs baseline: 1.1507x; 1.1507x over previous
import functools

import jax
import jax.numpy as jnp
from jax import lax
from jax.experimental import pallas as pl
from jax.experimental.pallas import tpu as pltpu
from jax.experimental.pallas import tpu_sc as plsc

BATCH = 16384
NUM_FIELDS = 26
EMBED_DIM = 16
TOTAL = BATCH * NUM_FIELDS

_info = plsc.get_sparse_core_info()
_NC, _NS = _info.num_cores, _info.num_subcores
_NW = _NC * _NS
_BATCH_PER_W = BATCH // _NW
_CB = 8
_CHUNK = _CB * NUM_FIELDS
_NSTEP = _BATCH_PER_W // _CB

_mesh = plsc.VectorSubcoreMesh(core_axis_name="c", subcore_axis_name="s")


@functools.partial(
    pl.kernel,
    mesh=_mesh,
    out_type=jax.ShapeDtypeStruct((BATCH, 32, 128), jnp.float32),
    scratch_types=[
        pltpu.VMEM((_CHUNK,), jnp.int32),
        pltpu.VMEM((_CHUNK,), jnp.int32),
        pltpu.VMEM((_CHUNK, EMBED_DIM), jnp.float32),
        pltpu.VMEM((_CHUNK, EMBED_DIM), jnp.float32),
        pltpu.SemaphoreType.DMA,
        pltpu.SemaphoreType.DMA,
        pltpu.SemaphoreType.DMA,
    ],
    compiler_params=pltpu.CompilerParams(use_tc_tiling_on_sc=False),
)
def _gather_rows(table_hbm, idx_hbm, out_hbm, idx_a, idx_b, rows_a, rows_b,
                 gsem_a, gsem_b, osem):
    wid = lax.axis_index("s") * _NC + lax.axis_index("c")

    def stores(rows_v, b0):
        return [
            pltpu.async_copy(
                rows_v.at[pl.ds(cb * NUM_FIELDS, NUM_FIELDS), :],
                out_hbm.at[b0 + cb, pl.ds(0, NUM_FIELDS),
                           pl.ds(0, EMBED_DIM)],
                osem)
            for cb in range(_CB)
        ]

    def step(k, carry):
        b0a = wid * _BATCH_PER_W + (2 * k) * _CB
        b0b = b0a + _CB
        pltpu.sync_copy(idx_hbm.at[pl.ds(b0a * NUM_FIELDS, _CHUNK)], idx_a)
        ga = pltpu.async_copy(table_hbm.at[idx_a], rows_a, gsem_a)
        pltpu.sync_copy(idx_hbm.at[pl.ds(b0b * NUM_FIELDS, _CHUNK)], idx_b)
        gb = pltpu.async_copy(table_hbm.at[idx_b], rows_b, gsem_b)
        ga.wait()
        cps_a = stores(rows_a, b0a)
        gb.wait()
        cps_b = stores(rows_b, b0b)
        for cp in cps_a + cps_b:
            cp.wait()
        return carry

    lax.fori_loop(0, _NSTEP // 2, step, 0)


def kernel(table, x):
    flat = x.reshape(TOTAL)
    out_pad = _gather_rows(table, flat)
    return out_pad[:, :NUM_FIELDS, :EMBED_DIM]


# speedup vs baseline: 1.1745x; 1.0207x over previous
import functools

import jax
import jax.numpy as jnp
from jax import lax
from jax.experimental import pallas as pl
from jax.experimental.pallas import tpu as pltpu
from jax.experimental.pallas import tpu_sc as plsc

BATCH = 16384
NUM_FIELDS = 26
EMBED_DIM = 16
TOTAL = BATCH * NUM_FIELDS

_info = plsc.get_sparse_core_info()
_NC, _NS = _info.num_cores, _info.num_subcores
_NW = _NC * _NS
_BATCH_PER_W = BATCH // _NW
_CB = 16
_CHUNK = _CB * NUM_FIELDS
_NSTEP = _BATCH_PER_W // _CB

_mesh = plsc.VectorSubcoreMesh(core_axis_name="c", subcore_axis_name="s")


@functools.partial(
    pl.kernel,
    mesh=_mesh,
    out_type=jax.ShapeDtypeStruct((BATCH, 32, 128), jnp.float32),
    scratch_types=[
        pltpu.VMEM((_CHUNK,), jnp.int32),
        pltpu.VMEM((_CHUNK,), jnp.int32),
        pltpu.VMEM((_CHUNK, EMBED_DIM), jnp.float32),
        pltpu.VMEM((_CHUNK, EMBED_DIM), jnp.float32),
        pltpu.SemaphoreType.DMA,
        pltpu.SemaphoreType.DMA,
        pltpu.SemaphoreType.DMA,
    ],
    compiler_params=pltpu.CompilerParams(use_tc_tiling_on_sc=False),
)
def _gather_rows(table_hbm, idx_hbm, out_hbm, idx_a, idx_b, rows_a, rows_b,
                 gsem_a, gsem_b, osem):
    wid = lax.axis_index("s") * _NC + lax.axis_index("c")

    def stores(rows_v, b0):
        return [
            pltpu.async_copy(
                rows_v.at[pl.ds(cb * NUM_FIELDS, NUM_FIELDS), :],
                out_hbm.at[b0 + cb, pl.ds(0, NUM_FIELDS),
                           pl.ds(0, EMBED_DIM)],
                osem)
            for cb in range(_CB)
        ]

    def step(k, carry):
        b0a = wid * _BATCH_PER_W + (2 * k) * _CB
        b0b = b0a + _CB
        pltpu.sync_copy(idx_hbm.at[pl.ds(b0a * NUM_FIELDS, _CHUNK)], idx_a)
        ga = pltpu.async_copy(table_hbm.at[idx_a], rows_a, gsem_a)
        pltpu.sync_copy(idx_hbm.at[pl.ds(b0b * NUM_FIELDS, _CHUNK)], idx_b)
        gb = pltpu.async_copy(table_hbm.at[idx_b], rows_b, gsem_b)
        ga.wait()
        cps_a = stores(rows_a, b0a)
        gb.wait()
        cps_b = stores(rows_b, b0b)
        for cp in cps_a + cps_b:
            cp.wait()
        return carry

    lax.fori_loop(0, _NSTEP // 2, step, 0)


def kernel(table, x):
    flat = x.reshape(TOTAL)
    out_pad = _gather_rows(table, flat)
    return out_pad[:, :NUM_FIELDS, :EMBED_DIM]
